# baseline (speedup 1.0000x reference)
import functools

import jax
import jax.numpy as jnp
from jax import lax
from jax.experimental import pallas as pl
from jax.experimental.pallas import tpu as pltpu
from jax.experimental.pallas import tpu_sc as plsc

NUM_EMBEDDINGS = 1024
EMBEDDING_DIM = 64
COMMITMENT_COST = 0.25
BM = 1024

_NC, _NS = 2, 16
_NW = _NC * _NS


def _vq_body(z_ref, zn_ref, en_ref, e_ref, idx_ref, loss_ref):
    i = pl.program_id(0)
    z = z_ref[0]
    e = e_ref[...]
    mm2 = lax.dot_general(-2.0 * z, e, (((1,), (1,)), ((), ())),
                          preferred_element_type=jnp.float32)
    zn = zn_ref[0].reshape(BM, 1)
    t = zn + en_ref[...]
    d = t + mm2
    minv = jnp.min(d, axis=1, keepdims=True)
    iota_f = lax.broadcasted_iota(jnp.int32, d.shape, 1).astype(jnp.float32)
    big = jnp.float32(NUM_EMBEDDINGS)
    idxf = jnp.min(jnp.where(d == minv, iota_f, big), axis=1, keepdims=True)
    idx_ref[...] = jnp.transpose(idxf, (1, 0)).astype(jnp.int32)[None]
    bs = jnp.sum(minv)[None, None]

    @pl.when(i == 0)
    def _init():
        loss_ref[...] = jnp.zeros_like(loss_ref)

    loss_ref[...] += bs


def _make_sc_gather(V, D, B):
    b_per_w = B // _NW
    mesh = plsc.VectorSubcoreMesh(core_axis_name="c", subcore_axis_name="s")

    nch = 8
    ch = b_per_w // nch

    @functools.partial(
        pl.kernel, mesh=mesh,
        out_type=jax.ShapeDtypeStruct((B, 128), jnp.float32),
        scratch_types=(
            [pltpu.VMEM((b_per_w,), jnp.int32)]
            + [pltpu.VMEM((ch, 128), jnp.float32)] * nch
            + [pltpu.SemaphoreType.DMA] * (2 * nch)
        ),
    )
    def gather_k(table_hbm, idx_hbm, out_hbm, idx_v, *bufs_sems):
        bufs = bufs_sems[:nch]
        gsems = bufs_sems[nch:2 * nch]
        wsems = bufs_sems[2 * nch:]
        wid = lax.axis_index("s") * _NC + lax.axis_index("c")
        base = wid * b_per_w
        pltpu.sync_copy(idx_hbm.at[base // BM, 0, pl.ds(base % BM, b_per_w)],
                        idx_v)
        gh = [pltpu.async_copy(table_hbm.at[idx_v.at[pl.ds(c * ch, ch)]],
                               bufs[c], gsems[c])
              for c in range(nch)]
        wh = []
        for c in range(nch):
            gh[c].wait()
            wh.append(pltpu.async_copy(
                bufs[c], out_hbm.at[pl.ds(base + c * ch, ch)], wsems[c]))
        for h in wh:
            h.wait()

    return gather_k


def kernel(z_e, embedding):
    B, T, C = z_e.shape
    M = B * T
    G = M // BM
    zn = jnp.sum(z_e.reshape(M, C) ** 2, axis=1).reshape(G, 1, BM)
    en = jnp.sum(embedding ** 2, axis=1, keepdims=True).T
    idx_out, loss_out = pl.pallas_call(
        _vq_body,
        grid=(G,),
        in_specs=[
            pl.BlockSpec((1, BM, C), lambda i: (i, 0, 0)),
            pl.BlockSpec((1, 1, BM), lambda i: (i, 0, 0)),
            pl.BlockSpec((1, NUM_EMBEDDINGS), lambda i: (0, 0)),
            pl.BlockSpec((NUM_EMBEDDINGS, C), lambda i: (0, 0)),
        ],
        out_specs=[
            pl.BlockSpec((1, 1, BM), lambda i: (i, 0, 0)),
            pl.BlockSpec((1, 1), lambda i: (0, 0)),
        ],
        out_shape=[
            jax.ShapeDtypeStruct((M // BM, 1, BM), jnp.int32),
            jax.ShapeDtypeStruct((1, 1), jnp.float32),
        ],
    )(z_e.reshape(G, BM, C), zn, en, embedding)
    emb_pad = jnp.pad(embedding, ((0, 0), (0, 128 - C)))
    zq128 = _make_sc_gather(NUM_EMBEDDINGS, C, M)(emb_pad, idx_out)
    zq = zq128[:, :C]
    commitment_loss = (COMMITMENT_COST / (M * C)) * loss_out[0, 0]
    return (zq.reshape(B, T, C), commitment_loss, idx_out.reshape(B, T))


# speedup vs baseline: 1.0167x; 1.0167x over previous
import functools

import jax
import jax.numpy as jnp
from jax import lax
from jax.experimental import pallas as pl
from jax.experimental.pallas import tpu as pltpu
from jax.experimental.pallas import tpu_sc as plsc

NUM_EMBEDDINGS = 1024
EMBEDDING_DIM = 64
COMMITMENT_COST = 0.25
BM = 1024

_NC, _NS = 2, 16
_NW = _NC * _NS


def _vq_body(z_ref, zn_ref, en_ref, e_ref, idx_ref, loss_ref):
    i = pl.program_id(0)
    z = z_ref[0]
    e = e_ref[...]
    mm2 = lax.dot_general(-2.0 * z, e, (((1,), (1,)), ((), ())),
                          preferred_element_type=jnp.float32)
    zn = zn_ref[0].reshape(BM, 1)
    t = zn + en_ref[...]
    d = t + mm2
    minv = jnp.min(d, axis=1, keepdims=True)
    iota_f = lax.broadcasted_iota(jnp.int32, d.shape, 1).astype(jnp.float32)
    big = jnp.float32(NUM_EMBEDDINGS)
    idxf = jnp.min(jnp.where(d == minv, iota_f, big), axis=1, keepdims=True)
    idx_ref[...] = jnp.transpose(idxf, (1, 0)).astype(jnp.int32)[None]
    bs = jnp.sum(minv)[None, None]

    @pl.when(i == 0)
    def _init():
        loss_ref[...] = jnp.zeros_like(loss_ref)

    loss_ref[...] += bs


def _make_sc_gather(V, D, B):
    b_per_w = B // _NW
    mesh = plsc.VectorSubcoreMesh(core_axis_name="c", subcore_axis_name="s")

    nch = 4
    ch = b_per_w // nch

    @functools.partial(
        pl.kernel, mesh=mesh,
        out_type=jax.ShapeDtypeStruct((B, 128), jnp.float32),
        scratch_types=(
            [pltpu.VMEM((b_per_w,), jnp.int32)]
            + [pltpu.VMEM((ch, 128), jnp.float32)] * nch
            + [pltpu.SemaphoreType.DMA] * (2 * nch)
        ),
    )
    def gather_k(table_hbm, idx_hbm, out_hbm, idx_v, *bufs_sems):
        bufs = bufs_sems[:nch]
        gsems = bufs_sems[nch:2 * nch]
        wsems = bufs_sems[2 * nch:]
        wid = lax.axis_index("s") * _NC + lax.axis_index("c")
        base = wid * b_per_w
        pltpu.sync_copy(idx_hbm.at[base // BM, 0, pl.ds(base % BM, b_per_w)],
                        idx_v)
        gh = [pltpu.async_copy(table_hbm.at[idx_v.at[pl.ds(c * ch, ch)]],
                               bufs[c], gsems[c])
              for c in range(nch)]
        wh = []
        for c in range(nch):
            gh[c].wait()
            wh.append(pltpu.async_copy(
                bufs[c], out_hbm.at[pl.ds(base + c * ch, ch)], wsems[c]))
        for h in wh:
            h.wait()

    return gather_k


def kernel(z_e, embedding):
    B, T, C = z_e.shape
    M = B * T
    G = M // BM
    zn = jnp.sum(z_e.reshape(M, C) ** 2, axis=1).reshape(G, 1, BM)
    en = jnp.sum(embedding ** 2, axis=1, keepdims=True).T
    idx_out, loss_out = pl.pallas_call(
        _vq_body,
        grid=(G,),
        in_specs=[
            pl.BlockSpec((1, BM, C), lambda i: (i, 0, 0)),
            pl.BlockSpec((1, 1, BM), lambda i: (i, 0, 0)),
            pl.BlockSpec((1, NUM_EMBEDDINGS), lambda i: (0, 0)),
            pl.BlockSpec((NUM_EMBEDDINGS, C), lambda i: (0, 0)),
        ],
        out_specs=[
            pl.BlockSpec((1, 1, BM), lambda i: (i, 0, 0)),
            pl.BlockSpec((1, 1), lambda i: (0, 0)),
        ],
        out_shape=[
            jax.ShapeDtypeStruct((M // BM, 1, BM), jnp.int32),
            jax.ShapeDtypeStruct((1, 1), jnp.float32),
        ],
    )(z_e.reshape(G, BM, C), zn, en, embedding)
    emb_pad = jnp.pad(embedding, ((0, 0), (0, 128 - C)))
    zq128 = _make_sc_gather(NUM_EMBEDDINGS, C, M)(emb_pad, idx_out)
    zq = zq128[:, :C]
    commitment_loss = (COMMITMENT_COST / (M * C)) * loss_out[0, 0]
    return (zq.reshape(B, T, C), commitment_loss, idx_out.reshape(B, T))


# speedup vs baseline: 1.0543x; 1.0370x over previous
import functools

import jax
import jax.numpy as jnp
from jax import lax
from jax.experimental import pallas as pl
from jax.experimental.pallas import tpu as pltpu
from jax.experimental.pallas import tpu_sc as plsc

NUM_EMBEDDINGS = 1024
EMBEDDING_DIM = 64
COMMITMENT_COST = 0.25
BM = 2048

_NC, _NS = 2, 16
_NW = _NC * _NS


def _vq_body(z_ref, zn_ref, en_ref, e_ref, idx_ref, loss_ref):
    i = pl.program_id(0)
    z = z_ref[0]
    e = e_ref[...]
    mm2 = lax.dot_general(-2.0 * z, e, (((1,), (1,)), ((), ())),
                          preferred_element_type=jnp.float32)
    zn = zn_ref[0].reshape(BM, 1)
    t = zn + en_ref[...]
    d = t + mm2
    minv = jnp.min(d, axis=1, keepdims=True)
    iota_f = lax.broadcasted_iota(jnp.int32, d.shape, 1).astype(jnp.float32)
    big = jnp.float32(NUM_EMBEDDINGS)
    idxf = jnp.min(jnp.where(d == minv, iota_f, big), axis=1, keepdims=True)
    idx_ref[...] = jnp.transpose(idxf, (1, 0)).astype(jnp.int32)[None]
    bs = jnp.sum(minv)[None, None]

    @pl.when(i == 0)
    def _init():
        loss_ref[...] = jnp.zeros_like(loss_ref)

    loss_ref[...] += bs


def _make_sc_gather(V, D, B):
    b_per_w = B // _NW
    mesh = plsc.VectorSubcoreMesh(core_axis_name="c", subcore_axis_name="s")

    nch = 4
    ch = b_per_w // nch

    @functools.partial(
        pl.kernel, mesh=mesh,
        out_type=jax.ShapeDtypeStruct((B, 128), jnp.float32),
        scratch_types=(
            [pltpu.VMEM((b_per_w,), jnp.int32)]
            + [pltpu.VMEM((ch, 128), jnp.float32)] * nch
            + [pltpu.SemaphoreType.DMA] * (2 * nch)
        ),
    )
    def gather_k(table_hbm, idx_hbm, out_hbm, idx_v, *bufs_sems):
        bufs = bufs_sems[:nch]
        gsems = bufs_sems[nch:2 * nch]
        wsems = bufs_sems[2 * nch:]
        wid = lax.axis_index("s") * _NC + lax.axis_index("c")
        base = wid * b_per_w
        pltpu.sync_copy(idx_hbm.at[base // BM, 0, pl.ds(base % BM, b_per_w)],
                        idx_v)
        gh = [pltpu.async_copy(table_hbm.at[idx_v.at[pl.ds(c * ch, ch)]],
                               bufs[c], gsems[c])
              for c in range(nch)]
        wh = []
        for c in range(nch):
            gh[c].wait()
            wh.append(pltpu.async_copy(
                bufs[c], out_hbm.at[pl.ds(base + c * ch, ch)], wsems[c]))
        for h in wh:
            h.wait()

    return gather_k


def kernel(z_e, embedding):
    B, T, C = z_e.shape
    M = B * T
    G = M // BM
    zn = jnp.sum(z_e.reshape(M, C) ** 2, axis=1).reshape(G, 1, BM)
    en = jnp.sum(embedding ** 2, axis=1, keepdims=True).T
    idx_out, loss_out = pl.pallas_call(
        _vq_body,
        grid=(G,),
        in_specs=[
            pl.BlockSpec((1, BM, C), lambda i: (i, 0, 0)),
            pl.BlockSpec((1, 1, BM), lambda i: (i, 0, 0)),
            pl.BlockSpec((1, NUM_EMBEDDINGS), lambda i: (0, 0)),
            pl.BlockSpec((NUM_EMBEDDINGS, C), lambda i: (0, 0)),
        ],
        out_specs=[
            pl.BlockSpec((1, 1, BM), lambda i: (i, 0, 0)),
            pl.BlockSpec((1, 1), lambda i: (0, 0)),
        ],
        out_shape=[
            jax.ShapeDtypeStruct((M // BM, 1, BM), jnp.int32),
            jax.ShapeDtypeStruct((1, 1), jnp.float32),
        ],
    )(z_e.reshape(G, BM, C), zn, en, embedding)
    emb_pad = jnp.pad(embedding, ((0, 0), (0, 128 - C)))
    zq128 = _make_sc_gather(NUM_EMBEDDINGS, C, M)(emb_pad, idx_out)
    zq = zq128[:, :C]
    commitment_loss = (COMMITMENT_COST / (M * C)) * loss_out[0, 0]
    return (zq.reshape(B, T, C), commitment_loss, idx_out.reshape(B, T))


# speedup vs baseline: 1.0594x; 1.0049x over previous
import functools

import jax
import jax.numpy as jnp
from jax import lax
from jax.experimental import pallas as pl
from jax.experimental.pallas import tpu as pltpu
from jax.experimental.pallas import tpu_sc as plsc

NUM_EMBEDDINGS = 1024
EMBEDDING_DIM = 64
COMMITMENT_COST = 0.25
BM = 4096

_NC, _NS = 2, 16
_NW = _NC * _NS


def _vq_body(z_ref, zn_ref, en_ref, e_ref, idx_ref, loss_ref):
    i = pl.program_id(0)
    z = z_ref[0]
    e = e_ref[...]
    mm2 = lax.dot_general(-2.0 * z, e, (((1,), (1,)), ((), ())),
                          preferred_element_type=jnp.float32)
    zn = zn_ref[0].reshape(BM, 1)
    t = zn + en_ref[...]
    d = t + mm2
    minv = jnp.min(d, axis=1, keepdims=True)
    iota_f = lax.broadcasted_iota(jnp.int32, d.shape, 1).astype(jnp.float32)
    big = jnp.float32(NUM_EMBEDDINGS)
    idxf = jnp.min(jnp.where(d == minv, iota_f, big), axis=1, keepdims=True)
    idx_ref[...] = jnp.transpose(idxf, (1, 0)).astype(jnp.int32)[None]
    bs = jnp.sum(minv)[None, None]

    @pl.when(i == 0)
    def _init():
        loss_ref[...] = jnp.zeros_like(loss_ref)

    loss_ref[...] += bs


def _make_sc_gather(V, D, B):
    b_per_w = B // _NW
    mesh = plsc.VectorSubcoreMesh(core_axis_name="c", subcore_axis_name="s")

    nch = 4
    ch = b_per_w // nch

    @functools.partial(
        pl.kernel, mesh=mesh,
        out_type=jax.ShapeDtypeStruct((B, 128), jnp.float32),
        scratch_types=(
            [pltpu.VMEM((b_per_w,), jnp.int32)]
            + [pltpu.VMEM((ch, 128), jnp.float32)] * nch
            + [pltpu.SemaphoreType.DMA] * (2 * nch)
        ),
    )
    def gather_k(table_hbm, idx_hbm, out_hbm, idx_v, *bufs_sems):
        bufs = bufs_sems[:nch]
        gsems = bufs_sems[nch:2 * nch]
        wsems = bufs_sems[2 * nch:]
        wid = lax.axis_index("s") * _NC + lax.axis_index("c")
        base = wid * b_per_w
        pltpu.sync_copy(idx_hbm.at[base // BM, 0, pl.ds(base % BM, b_per_w)],
                        idx_v)
        gh = [pltpu.async_copy(table_hbm.at[idx_v.at[pl.ds(c * ch, ch)]],
                               bufs[c], gsems[c])
              for c in range(nch)]
        wh = []
        for c in range(nch):
            gh[c].wait()
            wh.append(pltpu.async_copy(
                bufs[c], out_hbm.at[pl.ds(base + c * ch, ch)], wsems[c]))
        for h in wh:
            h.wait()

    return gather_k


def kernel(z_e, embedding):
    B, T, C = z_e.shape
    M = B * T
    G = M // BM
    zn = jnp.sum(z_e.reshape(M, C) ** 2, axis=1).reshape(G, 1, BM)
    en = jnp.sum(embedding ** 2, axis=1, keepdims=True).T
    idx_out, loss_out = pl.pallas_call(
        _vq_body,
        grid=(G,),
        in_specs=[
            pl.BlockSpec((1, BM, C), lambda i: (i, 0, 0)),
            pl.BlockSpec((1, 1, BM), lambda i: (i, 0, 0)),
            pl.BlockSpec((1, NUM_EMBEDDINGS), lambda i: (0, 0)),
            pl.BlockSpec((NUM_EMBEDDINGS, C), lambda i: (0, 0)),
        ],
        out_specs=[
            pl.BlockSpec((1, 1, BM), lambda i: (i, 0, 0)),
            pl.BlockSpec((1, 1), lambda i: (0, 0)),
        ],
        out_shape=[
            jax.ShapeDtypeStruct((M // BM, 1, BM), jnp.int32),
            jax.ShapeDtypeStruct((1, 1), jnp.float32),
        ],
    )(z_e.reshape(G, BM, C), zn, en, embedding)
    emb_pad = jnp.pad(embedding, ((0, 0), (0, 128 - C)))
    zq128 = _make_sc_gather(NUM_EMBEDDINGS, C, M)(emb_pad, idx_out)
    zq = zq128[:, :C]
    commitment_loss = (COMMITMENT_COST / (M * C)) * loss_out[0, 0]
    return (zq.reshape(B, T, C), commitment_loss, idx_out.reshape(B, T))
